# Initial kernel scaffold; baseline (speedup 1.0000x reference)
import jax
import jax.numpy as jnp
from jax.experimental import pallas as pl


def kernel(x, router_w, w1, w2):
    raise NotImplementedError("write your pallas kernel here")


# speedup vs baseline: 4.0992x; 4.0992x over previous
import functools

import jax
import jax.numpy as jnp
from jax.experimental import pallas as pl
from jax.experimental.pallas import tpu as pltpu

NUM_EXPERTS = 8
TOP_K = 2
D_MODEL = 768
FFN = 4 * D_MODEL
NF = 4
FB = FFN // NF


def _router_kernel(x_ref, rwt_ref, logits_ref, gates_ref):
    x = x_ref[...]
    logits = jnp.dot(x, rwt_ref[...], preferred_element_type=jnp.float32)
    logits_ref[...] = logits
    m = jnp.max(logits, axis=1, keepdims=True)
    ex = jnp.exp(logits - m)
    probs = ex / jnp.sum(ex, axis=1, keepdims=True)
    eidx = jax.lax.broadcasted_iota(jnp.int32, probs.shape, 1)
    m1 = jnp.max(probs, axis=1, keepdims=True)
    i1 = jnp.min(jnp.where(probs == m1, eidx, NUM_EXPERTS), axis=1,
                 keepdims=True)
    masked = jnp.where(eidx == i1, -jnp.inf, probs)
    m2 = jnp.max(masked, axis=1, keepdims=True)
    i2 = jnp.min(jnp.where(masked == m2, eidx, NUM_EXPERTS), axis=1,
                 keepdims=True)
    s = m1 + m2
    g = jnp.where(eidx == i1, m1 / s, 0.0) + jnp.where(eidx == i2, m2 / s, 0.0)
    gates_ref[...] = g


def _ffn_kernel(x_ref, w1_ref, w2_ref, gt_ref, out_ref):
    e = pl.program_id(0)
    j = pl.program_id(1)

    @pl.when(jnp.logical_and(e == 0, j == 0))
    def _():
        out_ref[...] = jnp.zeros_like(out_ref)

    x = x_ref[...].astype(jnp.bfloat16)
    w1 = w1_ref[...].astype(jnp.bfloat16)
    h = jnp.dot(x, w1, preferred_element_type=jnp.float32)
    h = h * 0.5 * (1.0 + jax.lax.erf(h * 0.7071067811865476))
    g = gt_ref[0]
    h = h * g[0, :, None]
    w2 = w2_ref[...].astype(jnp.bfloat16)
    y = jnp.dot(h.astype(jnp.bfloat16), w2, preferred_element_type=jnp.float32)
    out_ref[...] += y


@functools.partial(jax.jit, static_argnames=())
def kernel(x, router_w, w1, w2):
    B, S, D = x.shape
    T = B * S
    xf = x.reshape(T, D)

    logits, gates_t = pl.pallas_call(
        _router_kernel,
        out_shape=(
            jax.ShapeDtypeStruct((T, NUM_EXPERTS), jnp.float32),
            jax.ShapeDtypeStruct((T, NUM_EXPERTS), jnp.float32),
        ),
    )(xf, router_w.T)

    out = pl.pallas_call(
        _ffn_kernel,
        grid=(NUM_EXPERTS, NF),
        in_specs=[
            pl.BlockSpec((T, D), lambda e, j: (0, 0)),
            pl.BlockSpec((D, FB), lambda e, j: (0, e * NF + j)),
            pl.BlockSpec((FB, D), lambda e, j: (e * NF + j, 0)),
            pl.BlockSpec((1, 1, T), lambda e, j: (e, 0, 0)),
        ],
        out_specs=pl.BlockSpec((T, D), lambda e, j: (0, 0)),
        out_shape=jax.ShapeDtypeStruct((T, D), jnp.float32),
    )(xf, w1, w2, gates_t.T.reshape(NUM_EXPERTS, 1, T))

    return out.reshape(B, S, D), logits
